# baseline (speedup 1.0000x reference)
import functools

import jax
import jax.numpy as jnp
from jax import lax
from jax.experimental import pallas as pl
from jax.experimental.pallas import tpu as pltpu
from jax.experimental.pallas import tpu_sc as plsc

B = 16384
K = 32
NV = 1_000_000
NC = 2
NS = 16
NW = NC * NS
WIN = 768
NWIN = 41
SPAN = NWIN * WIN
NPAIR = 21
TAIL0 = 999_936

RATING_RANGE = 4.5
LOWEST_RATING = 0.5

_mesh = plsc.VectorSubcoreMesh(core_axis_name="c", subcore_axis_name="s")

_i32 = jnp.int32


def _iota():
    return lax.iota(_i32, 16)


def _splat(s):
    return jnp.full((16,), s, _i32)


@functools.partial(
    pl.kernel,
    mesh=_mesh,
    compiler_params=pltpu.CompilerParams(needs_layout_passes=False),
    out_type=[
        jax.ShapeDtypeStruct((NW * B, 128), jnp.float32),
        jax.ShapeDtypeStruct((B + 128, 128), jnp.float32),
        jax.ShapeDtypeStruct((NW * B, 128), jnp.float32),
        jax.ShapeDtypeStruct((B + 128, 128), jnp.float32),
    ],
    scratch_types=[
        pltpu.VMEM((B,), _i32),
        pltpu.VMEM((B,), _i32),
        pltpu.VMEM((K, WIN), jnp.float32),
        pltpu.VMEM((K, WIN), jnp.float32),
        pltpu.VMEM((K, 64), jnp.float32),
        pltpu.VMEM((32,), _i32),
        pltpu.VMEM((32,), _i32),
        pltpu.VMEM((144, 128), jnp.float32),
        pltpu.VMEM((128, 128), _i32),
        pltpu.SemaphoreType.DMA,
        pltpu.SemaphoreType.DMA,
    ],
)
def _scan_sc(users_hbm, items_hbm, ut_hbm, it_hbm, utail_hbm, itail_hbm,
             uslab_hbm, ux_hbm, islab_hbm, ix_hbm,
             wlc_v, wlj_v, win0_v, win1_v, twin_v, pc_v, pj_v, stg_v,
             jout_v, sem0, sem1):
    wid = lax.axis_index("s") * NC + lax.axis_index("c")
    lo = wid * SPAN
    hi = jnp.minimum(lo + SPAN, NV)
    iota = _iota()

    def extract(kvalid, ws_addr, wref, ecnt, fb, ck, slab_hbm):
        cv = pc_v[0:16] - ws_addr
        jv = pj_v[0:16]
        em = iota < _splat(kvalid)
        ep = _splat(ecnt) + iota
        el = ep - _splat(fb)
        plsc.store_scatter(jout_v, [ep >> 7, ep & 127], jv, mask=em)
        for f in range(K):
            vals = plsc.load_gather(wref, [_splat(f), cv], mask=em)
            plsc.store_scatter(stg_v, [el, _splat(f)], vals, mask=em)
        ecnt = ecnt + kvalid

        def flush(ops):
            ecnt2, fb2, ck2 = ops
            dst = pl.multiple_of((wid * 128 + ck2) * 128, 128)
            pltpu.sync_copy(stg_v.at[pl.ds(0, 128), :],
                            slab_hbm.at[pl.ds(dst, 128), :])
            for t in range(128):
                line = 128 + t // 8
                col = (t % 8) * 16
                v = plsc.load_gather(stg_v, [_splat(line), col + iota])
                plsc.store_scatter(stg_v, [_splat(line - 128), col + iota], v)
            return ecnt2, fb2 + 128, ck2 + 1

        return lax.cond(ecnt - fb >= 128, flush, lambda o: o, (ecnt, fb, ck))

    def do_window(cnt, state, ws_addr, mlo, mhi, wref, slab_hbm):
        ngrp = (cnt + 15) // 16

        def grp(g, st):
            ecnt, pcnt, fb, ck = st
            gi = g * 16 + iota
            c = plsc.load_gather(wlc_v, [gi])
            j = plsc.load_gather(wlj_v, [gi])
            m = (gi < _splat(cnt)) & (c >= _splat(mlo)) & (c < _splat(mhi))
            inc = plsc.cumsum(m.astype(_i32))
            pos = _splat(pcnt) + inc - 1
            plsc.store_scatter(pc_v, [pos], c, mask=m)
            plsc.store_scatter(pj_v, [pos], j, mask=m)
            npend = pcnt + plsc.all_reduce_population_count(m)[0]

            def do_ex(ops):
                ecnt2, fb2, ck2 = ops
                ecnt3, fb3, ck3 = extract(16, ws_addr, wref, ecnt2, fb2, ck2,
                                          slab_hbm)
                sc = plsc.load_gather(pc_v, [iota + 16])
                sj = plsc.load_gather(pj_v, [iota + 16])
                pc_v[0:16] = sc
                pj_v[0:16] = sj
                return ecnt3, fb3, ck3

            ecnt, fb, ck = lax.cond(npend >= 16, do_ex, lambda o: o,
                                    (ecnt, fb, ck))
            npend = jnp.where(npend >= 16, npend - 16, npend)
            return ecnt, npend, fb, ck

        ecnt, pcnt, fb, ck = lax.fori_loop(0, ngrp, grp, state)

        def drain(ops):
            ecnt2, pcnt2, fb2, ck2 = ops
            ecnt3, fb3, ck3 = extract(pcnt2, ws_addr, wref, ecnt2, fb2, ck2,
                                      slab_hbm)
            return ecnt3, 0, fb3, ck3

        return lax.cond(pcnt > 0, drain, lambda o: o, (ecnt, pcnt, fb, ck))

    def phase(list_hbm, tab_hbm, tail_hbm, slab_hbm, out_hbm):
        pltpu.sync_copy(list_hbm, jout_v)
        pltpu.sync_copy(tail_hbm, twin_v)

        def fire(slot, buf, sem):
            s = lo + slot * WIN
            s_eff = jnp.where(s + WIN <= NV, s, 0)
            pltpu.async_copy(
                tab_hbm.at[:, pl.ds(pl.multiple_of(s_eff, 128), WIN)], buf,
                sem)

        def bwait(buf, sem):
            pltpu.make_async_copy(tab_hbm.at[:, pl.ds(0, WIN)], buf,
                                  sem).wait()

        fire(0, win0_v, sem0)

        def bucket(g, cnt):
            gi = g * 16 + iota
            idx = plsc.load_gather(jout_v, [gi >> 7, gi & 127])
            m = (idx >= _splat(lo)) & (idx < _splat(hi))
            inc = plsc.cumsum(m.astype(_i32))
            pos = _splat(cnt) + inc - 1
            plsc.store_scatter(wlc_v, [pos], idx, mask=m)
            plsc.store_scatter(wlj_v, [pos], gi, mask=m)
            return cnt + plsc.all_reduce_population_count(m)[0]

        cnt = lax.fori_loop(0, B // 16, bucket, jnp.int32(0))

        def jinit(g, _):
            plsc.store_scatter(jout_v, [_splat(g >> 3), (g & 7) * 16 + iota],
                               _splat(B))
            return _

        lax.fori_loop(0, ((cnt + 127) // 128) * 8, jinit, 0)

        def process(slot, buf, st):
            s = lo + slot * WIN
            valid = s + WIN <= NV
            mlo = jnp.where(valid, s, -1)
            mhi = jnp.where(valid, s + WIN, -1)
            return do_window(cnt, st, s, mlo, mhi, buf, slab_hbm)

        def pair(t2, st):
            a = 2 * t2
            fire(a + 1, win1_v, sem1)
            bwait(win0_v, sem0)
            st = process(a, win0_v, st)
            fire(a + 2, win0_v, sem0)
            bwait(win1_v, sem1)
            st = process(a + 1, win1_v, st)
            return st

        state = lax.fori_loop(0, NPAIR, pair,
                              (jnp.int32(0), jnp.int32(0), jnp.int32(0),
                               jnp.int32(0)))
        bwait(win0_v, sem0)

        state = do_window(cnt, state, TAIL0, TAIL0, NV, twin_v, slab_hbm)

        ecnt, _, fb, ck = state

        def ffl(ops):
            fb2, ck2 = ops
            dst = pl.multiple_of((wid * 128 + ck2) * 128, 128)
            pltpu.sync_copy(stg_v.at[pl.ds(0, 128), :],
                            slab_hbm.at[pl.ds(dst, 128), :])
            return fb2, ck2

        lax.cond(ecnt > fb, ffl, lambda o: o, (fb, ck))

        def un(c, _):
            src = pl.multiple_of((wid * 128 + c) * 128, 128)
            pltpu.sync_copy(slab_hbm.at[pl.ds(src, 128), :],
                            stg_v.at[pl.ds(0, 128), :])
            pltpu.async_copy(stg_v.at[pl.ds(0, 128), :],
                             out_hbm.at[jout_v.at[c]], sem0).wait()
            return _

        lax.fori_loop(0, (ecnt + 127) // 128, un, 0)

    phase(users_hbm, ut_hbm, utail_hbm, uslab_hbm, ux_hbm)
    phase(items_hbm, it_hbm, itail_hbm, islab_hbm, ix_hbm)


def _mlp_body(ux_ref, ix_ref, w1u_ref, w1i_ref, b1_ref, w2_ref, b2_ref,
              woutt_ref, bout_ref, out_ref):
    h = jnp.dot(ux_ref[:, :K], w1u_ref[...],
                preferred_element_type=jnp.float32,
                precision=lax.Precision.HIGHEST)
    h = h + jnp.dot(ix_ref[:, :K], w1i_ref[...],
                    preferred_element_type=jnp.float32,
                    precision=lax.Precision.HIGHEST)
    h = jnp.maximum(h + b1_ref[...], 0.0)
    h = jnp.maximum(
        jnp.dot(h, w2_ref[...], preferred_element_type=jnp.float32,
                precision=lax.Precision.HIGHEST) + b2_ref[...], 0.0)
    o = jnp.sum(h * woutt_ref[...], axis=1, keepdims=True) + bout_ref[...]
    out_ref[...] = jax.nn.sigmoid(o) * RATING_RANGE + LOWEST_RATING


def kernel(users, items, U, I, W1, b1, W2, b2, Wout, bout):
    users2d = users.astype(_i32).reshape(128, 128)
    items2d = items.astype(_i32).reshape(128, 128)
    ut = U.T
    it = I.T
    utail = U[TAIL0:, :].T
    itail = I[TAIL0:, :].T
    _, ux, _, ix = _scan_sc(users2d, items2d, ut, it, utail, itail)

    w1u = W1[:K, :]
    w1i = W1[K:, :]
    bb = 4096
    full = lambda shape: pl.BlockSpec(shape, lambda i: (0, 0))
    out = pl.pallas_call(
        _mlp_body,
        grid=(B // bb,),
        in_specs=[
            pl.BlockSpec((bb, 128), lambda i: (i, 0)),
            pl.BlockSpec((bb, 128), lambda i: (i, 0)),
            full((K, 64)),
            full((K, 64)),
            full((1, 64)),
            full((64, K)),
            full((1, K)),
            full((1, K)),
            full((1, 1)),
        ],
        out_specs=pl.BlockSpec((bb, 1), lambda i: (i, 0)),
        out_shape=jax.ShapeDtypeStruct((B, 1), jnp.float32),
    )(ux, ix, w1u, w1i, b1.reshape(1, -1), W2, b2.reshape(1, -1),
      Wout.reshape(1, -1), bout.reshape(1, 1))
    return out


# speedup vs baseline: 1.5361x; 1.5361x over previous
import functools

import jax
import jax.numpy as jnp
from jax import lax
from jax.experimental import pallas as pl
from jax.experimental.pallas import tpu as pltpu
from jax.experimental.pallas import tpu_sc as plsc

B = 16384
K = 32
NV = 1_000_000
NC = 2
NS = 16
NW = NC * NS
WIN = 768
NWIN = 41
SPAN = NWIN * WIN
NPAIR = 21
TAIL0 = 999_936

RATING_RANGE = 4.5
LOWEST_RATING = 0.5

_mesh = plsc.VectorSubcoreMesh(core_axis_name="c", subcore_axis_name="s")

_i32 = jnp.int32


def _iota():
    return lax.iota(_i32, 16)


def _splat(s):
    return jnp.full((16,), s, _i32)


@functools.partial(
    pl.kernel,
    mesh=_mesh,
    compiler_params=pltpu.CompilerParams(needs_layout_passes=False),
    out_type=[
        jax.ShapeDtypeStruct((NW * 128 * 32, 128), jnp.float32),
        jax.ShapeDtypeStruct((NW, 128, 128), _i32),
        jax.ShapeDtypeStruct((NW * 8, 128), _i32),
        jax.ShapeDtypeStruct((NW * 128 * 32, 128), jnp.float32),
        jax.ShapeDtypeStruct((NW, 128, 128), _i32),
        jax.ShapeDtypeStruct((NW * 8, 128), _i32),
    ],
    scratch_types=[
        pltpu.VMEM((B,), _i32),
        pltpu.VMEM((B,), _i32),
        pltpu.VMEM((B,), _i32),
        pltpu.VMEM((K, WIN), jnp.float32),
        pltpu.VMEM((K, WIN), jnp.float32),
        pltpu.VMEM((K, 64), jnp.float32),
        pltpu.VMEM((32,), _i32),
        pltpu.VMEM((32,), _i32),
        pltpu.VMEM((40, 128), jnp.float32),
        pltpu.VMEM((128, 128), _i32),
        pltpu.VMEM((8, 128), _i32),
        pltpu.SemaphoreType.DMA,
        pltpu.SemaphoreType.DMA,
    ],
)
def _scan_sc(users_hbm, items_hbm, ut_hbm, it_hbm, utail_hbm, itail_hbm,
             urows_hbm, ujpos_hbm, ucnt_hbm, irows_hbm, ijpos_hbm, icnt_hbm,
             gl_v, wlc_v, wlj_v, win0_v, win1_v, twin_v, pc_v, pj_v, stg_v,
             jout_v, cbuf_v, sem0, sem1):
    wid = lax.axis_index("s") * NC + lax.axis_index("c")
    lo = wid * SPAN
    hi = jnp.minimum(lo + SPAN, NV)
    iota = _iota()

    def extract(kvalid, ws_addr, wref, ecnt, fb, ck, rows_hbm):
        cv = pc_v[0:16] - ws_addr
        jv = pj_v[0:16]
        em = iota < _splat(kvalid)
        ep = _splat(ecnt) + iota
        el = ep - _splat(fb)
        plsc.store_scatter(jout_v, [ep >> 7, ep & 127], jv, mask=em)
        for f in range(K):
            vals = plsc.load_gather(wref, [_splat(f), cv], mask=em)
            w = el * 32 + f
            plsc.store_scatter(stg_v, [w >> 7, w & 127], vals, mask=em)
        ecnt = ecnt + kvalid

        def flush(ops):
            ecnt2, fb2, ck2 = ops
            dst_row = pl.multiple_of((wid * 128 + ck2) * 32, 32)
            pltpu.sync_copy(stg_v.at[pl.ds(0, 32), :],
                            rows_hbm.at[pl.ds(dst_row, 32), :])
            for t in range(32):
                line = 32 + t // 8
                col = (t % 8) * 16
                v = plsc.load_gather(stg_v, [_splat(line), col + iota])
                plsc.store_scatter(stg_v, [_splat(line - 32), col + iota], v)
            return ecnt2, fb2 + 128, ck2 + 1

        return lax.cond(ecnt - fb >= 128, flush, lambda o: o, (ecnt, fb, ck))

    def do_window(cnt, state, ws_addr, mlo, mhi, wref, rows_hbm):
        ngrp = (cnt + 15) // 16

        def grp(g, st):
            ecnt, pcnt, fb, ck = st
            gi = g * 16 + iota
            c = plsc.load_gather(wlc_v, [gi])
            j = plsc.load_gather(wlj_v, [gi])
            m = (gi < _splat(cnt)) & (c >= _splat(mlo)) & (c < _splat(mhi))
            inc = plsc.cumsum(m.astype(_i32))
            pos = _splat(pcnt) + inc - 1
            plsc.store_scatter(pc_v, [pos], c, mask=m)
            plsc.store_scatter(pj_v, [pos], j, mask=m)
            npend = pcnt + plsc.all_reduce_population_count(m)[0]

            def do_ex(ops):
                ecnt2, fb2, ck2 = ops
                ecnt3, fb3, ck3 = extract(16, ws_addr, wref, ecnt2, fb2, ck2,
                                          rows_hbm)
                sc = plsc.load_gather(pc_v, [iota + 16])
                sj = plsc.load_gather(pj_v, [iota + 16])
                pc_v[0:16] = sc
                pj_v[0:16] = sj
                return ecnt3, fb3, ck3

            ecnt, fb, ck = lax.cond(npend >= 16, do_ex, lambda o: o,
                                    (ecnt, fb, ck))
            npend = jnp.where(npend >= 16, npend - 16, npend)
            return ecnt, npend, fb, ck

        ecnt, pcnt, fb, ck = lax.fori_loop(0, ngrp, grp, state)

        def drain(ops):
            ecnt2, pcnt2, fb2, ck2 = ops
            ecnt3, fb3, ck3 = extract(pcnt2, ws_addr, wref, ecnt2, fb2, ck2,
                                      rows_hbm)
            return ecnt3, 0, fb3, ck3

        return lax.cond(pcnt > 0, drain, lambda o: o, (ecnt, pcnt, fb, ck))

    def phase(list_hbm, tab_hbm, tail_hbm, rows_hbm, jpos_hbm, cnt_hbm):
        pltpu.sync_copy(list_hbm, gl_v)
        pltpu.sync_copy(tail_hbm, twin_v)

        def bucket(g, cnt):
            gi = g * 16 + iota
            idx = plsc.load_gather(gl_v, [gi])
            m = (idx >= _splat(lo)) & (idx < _splat(hi))
            inc = plsc.cumsum(m.astype(_i32))
            pos = _splat(cnt) + inc - 1
            plsc.store_scatter(wlc_v, [pos], idx, mask=m)
            plsc.store_scatter(wlj_v, [pos], gi, mask=m)
            return cnt + plsc.all_reduce_population_count(m)[0]

        cnt = lax.fori_loop(0, B // 16, bucket, jnp.int32(0))

        def jinit(g, _):
            plsc.store_scatter(jout_v, [_splat(g >> 3), (g & 7) * 16 + iota],
                               _splat(B))
            return _

        lax.fori_loop(0, ((cnt + 127) // 128) * 8, jinit, 0)

        def fire(slot, buf, sem):
            s = lo + slot * WIN
            s_eff = jnp.where(s + WIN <= NV, s, 0)
            pltpu.async_copy(
                tab_hbm.at[:, pl.ds(pl.multiple_of(s_eff, 128), WIN)], buf,
                sem)

        def bwait(buf, sem):
            pltpu.make_async_copy(tab_hbm.at[:, pl.ds(0, WIN)], buf,
                                  sem).wait()

        def process(slot, buf, st):
            s = lo + slot * WIN
            valid = s + WIN <= NV
            mlo = jnp.where(valid, s, -1)
            mhi = jnp.where(valid, s + WIN, -1)
            return do_window(cnt, st, s, mlo, mhi, buf, rows_hbm)

        fire(0, win0_v, sem0)

        def pair(t2, st):
            a = 2 * t2
            fire(a + 1, win1_v, sem1)
            bwait(win0_v, sem0)
            st = process(a, win0_v, st)
            fire(a + 2, win0_v, sem0)
            bwait(win1_v, sem1)
            st = process(a + 1, win1_v, st)
            return st

        state = lax.fori_loop(0, NPAIR, pair,
                              (jnp.int32(0), jnp.int32(0), jnp.int32(0),
                               jnp.int32(0)))
        bwait(win0_v, sem0)

        state = do_window(cnt, state, TAIL0, TAIL0, NV, twin_v, rows_hbm)

        ecnt, _, fb, ck = state

        def ffl(ops):
            fb2, ck2 = ops
            dst_row = pl.multiple_of((wid * 128 + ck2) * 32, 32)
            pltpu.sync_copy(stg_v.at[pl.ds(0, 32), :],
                            rows_hbm.at[pl.ds(dst_row, 32), :])
            return fb2, ck2

        lax.cond(ecnt > fb, ffl, lambda o: o, (fb, ck))

        pltpu.sync_copy(jout_v, jpos_hbm.at[wid])

        def cinit(g, _):
            plsc.store_scatter(cbuf_v, [_splat(g >> 3), (g & 7) * 16 + iota],
                               _splat(ecnt))
            return _

        lax.fori_loop(0, 64, cinit, 0)
        pltpu.sync_copy(cbuf_v,
                        cnt_hbm.at[pl.ds(pl.multiple_of(wid * 8, 8), 8), :])

    phase(users_hbm, ut_hbm, utail_hbm, urows_hbm, ujpos_hbm, ucnt_hbm)
    phase(items_hbm, it_hbm, itail_hbm, irows_hbm, ijpos_hbm, icnt_hbm)


@functools.partial(
    pl.kernel,
    mesh=_mesh,
    compiler_params=pltpu.CompilerParams(use_tc_tiling_on_sc=False),
    out_type=[
        jax.ShapeDtypeStruct((B + 128, K), jnp.float32),
        jax.ShapeDtypeStruct((B + 128, K), jnp.float32),
    ],
    scratch_types=[
        pltpu.VMEM((128, 128), _i32),
        pltpu.VMEM((8, 128), _i32),
        pltpu.VMEM((128, K), jnp.float32),
        pltpu.SemaphoreType.DMA,
    ],
)
def _unscramble_sc(urows_hbm, ujpos_hbm, ucnt_hbm, irows_hbm, ijpos_hbm,
                   icnt_hbm, ux_hbm, ix_hbm, jv_v, cb_v, rb_v, sem):
    wid = lax.axis_index("s") * NC + lax.axis_index("c")

    def phase(rows_hbm, jpos_hbm, cnt_hbm, out_hbm):
        pltpu.sync_copy(cnt_hbm.at[pl.ds(wid * 8, 8)], cb_v)
        cnt = cb_v[0, :][0]
        pltpu.sync_copy(jpos_hbm.at[wid], jv_v)
        nch = (cnt + 127) // 128

        def chunk(c, _):
            pltpu.sync_copy(rows_hbm.at[pl.ds(wid * B + c * 128, 128)], rb_v)
            pltpu.async_copy(rb_v, out_hbm.at[jv_v.at[c]], sem).wait()
            return _

        lax.fori_loop(0, nch, chunk, 0)

    phase(urows_hbm, ujpos_hbm, ucnt_hbm, ux_hbm)
    phase(irows_hbm, ijpos_hbm, icnt_hbm, ix_hbm)


def _mlp_body(ux_ref, ix_ref, w1u_ref, w1i_ref, b1_ref, w2_ref, b2_ref,
              woutt_ref, bout_ref, out_ref):
    h = jnp.dot(ux_ref[...], w1u_ref[...],
                preferred_element_type=jnp.float32,
                precision=lax.Precision.HIGHEST)
    h = h + jnp.dot(ix_ref[...], w1i_ref[...],
                    preferred_element_type=jnp.float32,
                    precision=lax.Precision.HIGHEST)
    h = jnp.maximum(h + b1_ref[...], 0.0)
    h = jnp.maximum(
        jnp.dot(h, w2_ref[...], preferred_element_type=jnp.float32,
                precision=lax.Precision.HIGHEST) + b2_ref[...], 0.0)
    o = jnp.sum(h * woutt_ref[...], axis=1, keepdims=True) + bout_ref[...]
    out_ref[...] = jax.nn.sigmoid(o) * RATING_RANGE + LOWEST_RATING


def kernel(users, items, U, I, W1, b1, W2, b2, Wout, bout):
    users = users.astype(_i32)
    items = items.astype(_i32)
    ut = U.T
    it = I.T
    utail = U[TAIL0:, :].T
    itail = I[TAIL0:, :].T
    ur, uj, uc, ir, ij, ic = _scan_sc(users, items, ut, it, utail, itail)
    ux, ix = _unscramble_sc(ur.reshape(NW * B, K), uj, uc,
                            ir.reshape(NW * B, K), ij, ic)

    w1u = W1[:K, :]
    w1i = W1[K:, :]
    bb = 4096
    full = lambda shape: pl.BlockSpec(shape, lambda i: (0, 0))
    out = pl.pallas_call(
        _mlp_body,
        grid=(B // bb,),
        in_specs=[
            pl.BlockSpec((bb, K), lambda i: (i, 0)),
            pl.BlockSpec((bb, K), lambda i: (i, 0)),
            full((K, 64)),
            full((K, 64)),
            full((1, 64)),
            full((64, K)),
            full((1, K)),
            full((1, K)),
            full((1, 1)),
        ],
        out_specs=pl.BlockSpec((bb, 1), lambda i: (i, 0)),
        out_shape=jax.ShapeDtypeStruct((B, 1), jnp.float32),
    )(ux, ix, w1u, w1i, b1.reshape(1, -1), W2, b2.reshape(1, -1),
      Wout.reshape(1, -1), bout.reshape(1, 1))
    return out


# speedup vs baseline: 1.6926x; 1.1019x over previous
import functools

import jax
import jax.numpy as jnp
from jax import lax
from jax.experimental import pallas as pl
from jax.experimental.pallas import tpu as pltpu
from jax.experimental.pallas import tpu_sc as plsc

B = 16384
K = 32
NV = 1_000_000
NC = 2
NS = 16
NW = NC * NS
WIN = 768
NWIN = 41
SPAN = NWIN * WIN
NPAIR = 21
TAIL0 = 999_936

RATING_RANGE = 4.5
LOWEST_RATING = 0.5

_mesh = plsc.VectorSubcoreMesh(core_axis_name="c", subcore_axis_name="s")

_i32 = jnp.int32


def _iota():
    return lax.iota(_i32, 16)


def _splat(s):
    return jnp.full((16,), s, _i32)


@functools.partial(
    pl.kernel,
    mesh=_mesh,
    compiler_params=pltpu.CompilerParams(needs_layout_passes=False),
    out_type=[
        jax.ShapeDtypeStruct((NW * 128 * 32, 128), jnp.float32),
        jax.ShapeDtypeStruct((NW, 128, 128), _i32),
        jax.ShapeDtypeStruct((NW * 8, 128), _i32),
        jax.ShapeDtypeStruct((NW * 128 * 32, 128), jnp.float32),
        jax.ShapeDtypeStruct((NW, 128, 128), _i32),
        jax.ShapeDtypeStruct((NW * 8, 128), _i32),
    ],
    scratch_types=[
        pltpu.VMEM((B,), _i32),
        pltpu.VMEM((B,), _i32),
        pltpu.VMEM((B,), _i32),
        pltpu.VMEM((K, WIN), jnp.float32),
        pltpu.VMEM((K, WIN), jnp.float32),
        pltpu.VMEM((K, 64), jnp.float32),
        pltpu.VMEM((32,), _i32),
        pltpu.VMEM((32,), _i32),
        pltpu.VMEM((40, 128), jnp.float32),
        pltpu.VMEM((128, 128), _i32),
        pltpu.VMEM((8, 128), _i32),
        pltpu.SemaphoreType.DMA,
        pltpu.SemaphoreType.DMA,
    ],
)
def _scan_sc(users_hbm, items_hbm, ut_hbm, it_hbm, utail_hbm, itail_hbm,
             urows_hbm, ujpos_hbm, ucnt_hbm, irows_hbm, ijpos_hbm, icnt_hbm,
             gl_v, wlc_v, wlj_v, win0_v, win1_v, twin_v, pc_v, pj_v, stg_v,
             jout_v, cbuf_v, sem0, sem1):
    wid = lax.axis_index("s") * NC + lax.axis_index("c")
    lo = wid * SPAN
    hi = jnp.minimum(lo + SPAN, NV)
    iota = _iota()

    def extract(kvalid, ws_addr, wref, ecnt, fb, ck, rows_hbm):
        cv = pc_v[0:16] - ws_addr
        jv = pj_v[0:16]
        em = iota < _splat(kvalid)
        ep = _splat(ecnt) + iota
        el = ep - _splat(fb)
        plsc.store_scatter(jout_v, [ep >> 7, ep & 127], jv, mask=em)
        for f in range(K):
            vals = plsc.load_gather(wref, [_splat(f), cv], mask=em)
            w = el * 32 + f
            plsc.store_scatter(stg_v, [w >> 7, w & 127], vals, mask=em)
        ecnt = ecnt + kvalid

        def flush(ops):
            ecnt2, fb2, ck2 = ops
            dst_row = pl.multiple_of((wid * 128 + ck2) * 32, 32)
            pltpu.sync_copy(stg_v.at[pl.ds(0, 32), :],
                            rows_hbm.at[pl.ds(dst_row, 32), :])
            for t in range(32):
                line = 32 + t // 8
                col = (t % 8) * 16
                v = plsc.load_gather(stg_v, [_splat(line), col + iota])
                plsc.store_scatter(stg_v, [_splat(line - 32), col + iota], v)
            return ecnt2, fb2 + 128, ck2 + 1

        return lax.cond(ecnt - fb >= 128, flush, lambda o: o, (ecnt, fb, ck))

    def do_window(cnt, state, ws_addr, mlo, mhi, wref, rows_hbm):
        ngrp = (cnt + 15) // 16

        def grp(g, st):
            ecnt, pcnt, fb, ck = st
            gi = g * 16 + iota
            c = plsc.load_gather(wlc_v, [gi])
            j = plsc.load_gather(wlj_v, [gi])
            m = (gi < _splat(cnt)) & (c >= _splat(mlo)) & (c < _splat(mhi))
            inc = plsc.cumsum(m.astype(_i32))
            pos = _splat(pcnt) + inc - 1
            plsc.store_scatter(pc_v, [pos], c, mask=m)
            plsc.store_scatter(pj_v, [pos], j, mask=m)
            npend = pcnt + plsc.all_reduce_population_count(m)[0]

            def do_ex(ops):
                ecnt2, fb2, ck2 = ops
                ecnt3, fb3, ck3 = extract(16, ws_addr, wref, ecnt2, fb2, ck2,
                                          rows_hbm)
                sc = plsc.load_gather(pc_v, [iota + 16])
                sj = plsc.load_gather(pj_v, [iota + 16])
                pc_v[0:16] = sc
                pj_v[0:16] = sj
                return ecnt3, fb3, ck3

            ecnt, fb, ck = lax.cond(npend >= 16, do_ex, lambda o: o,
                                    (ecnt, fb, ck))
            npend = jnp.where(npend >= 16, npend - 16, npend)
            return ecnt, npend, fb, ck

        ecnt, pcnt, fb, ck = lax.fori_loop(0, ngrp, grp, state)

        def drain(ops):
            ecnt2, pcnt2, fb2, ck2 = ops
            ecnt3, fb3, ck3 = extract(pcnt2, ws_addr, wref, ecnt2, fb2, ck2,
                                      rows_hbm)
            return ecnt3, 0, fb3, ck3

        return lax.cond(pcnt > 0, drain, lambda o: o, (ecnt, pcnt, fb, ck))

    def phase(list_hbm, tab_hbm, tail_hbm, rows_hbm, jpos_hbm, cnt_hbm):
        pltpu.sync_copy(list_hbm, gl_v)
        pltpu.sync_copy(tail_hbm, twin_v)

        def bucket(g, cnt):
            gi = g * 16 + iota
            idx = plsc.load_gather(gl_v, [gi])
            m = (idx >= _splat(lo)) & (idx < _splat(hi))
            inc = plsc.cumsum(m.astype(_i32))
            pos = _splat(cnt) + inc - 1
            plsc.store_scatter(wlc_v, [pos], idx, mask=m)
            plsc.store_scatter(wlj_v, [pos], gi, mask=m)
            return cnt + plsc.all_reduce_population_count(m)[0]

        cnt = lax.fori_loop(0, B // 16, bucket, jnp.int32(0))

        def jinit(g, _):
            plsc.store_scatter(jout_v, [_splat(g >> 3), (g & 7) * 16 + iota],
                               _splat(B))
            return _

        lax.fori_loop(0, ((cnt + 127) // 128) * 8, jinit, 0)

        def fire(slot, buf, sem):
            s = lo + slot * WIN
            s_eff = jnp.where(s + WIN <= NV, s, 0)
            pltpu.async_copy(
                tab_hbm.at[:, pl.ds(pl.multiple_of(s_eff, 128), WIN)], buf,
                sem)

        def bwait(buf, sem):
            pltpu.make_async_copy(tab_hbm.at[:, pl.ds(0, WIN)], buf,
                                  sem).wait()

        def process(slot, buf, st):
            s = lo + slot * WIN
            valid = s + WIN <= NV
            mlo = jnp.where(valid, s, -1)
            mhi = jnp.where(valid, s + WIN, -1)
            return do_window(cnt, st, s, mlo, mhi, buf, rows_hbm)

        fire(0, win0_v, sem0)

        def pair(t2, st):
            a = 2 * t2
            fire(a + 1, win1_v, sem1)
            bwait(win0_v, sem0)
            st = process(a, win0_v, st)
            fire(a + 2, win0_v, sem0)
            bwait(win1_v, sem1)
            st = process(a + 1, win1_v, st)
            return st

        state = lax.fori_loop(0, NPAIR, pair,
                              (jnp.int32(0), jnp.int32(0), jnp.int32(0),
                               jnp.int32(0)))
        bwait(win0_v, sem0)

        state = do_window(cnt, state, TAIL0, TAIL0, NV, twin_v, rows_hbm)

        ecnt, _, fb, ck = state

        def ffl(ops):
            fb2, ck2 = ops
            dst_row = pl.multiple_of((wid * 128 + ck2) * 32, 32)
            pltpu.sync_copy(stg_v.at[pl.ds(0, 32), :],
                            rows_hbm.at[pl.ds(dst_row, 32), :])
            return fb2, ck2

        lax.cond(ecnt > fb, ffl, lambda o: o, (fb, ck))

        pltpu.sync_copy(jout_v, jpos_hbm.at[wid])

        def cinit(g, _):
            plsc.store_scatter(cbuf_v, [_splat(g >> 3), (g & 7) * 16 + iota],
                               _splat(ecnt))
            return _

        lax.fori_loop(0, 64, cinit, 0)
        pltpu.sync_copy(cbuf_v,
                        cnt_hbm.at[pl.ds(pl.multiple_of(wid * 8, 8), 8), :])

    phase(users_hbm, ut_hbm, utail_hbm, urows_hbm, ujpos_hbm, ucnt_hbm)
    phase(items_hbm, it_hbm, itail_hbm, irows_hbm, ijpos_hbm, icnt_hbm)


@functools.partial(
    pl.kernel,
    mesh=_mesh,
    compiler_params=pltpu.CompilerParams(use_tc_tiling_on_sc=False),
    out_type=[
        jax.ShapeDtypeStruct((B + 128, K), jnp.float32),
        jax.ShapeDtypeStruct((B + 128, K), jnp.float32),
    ],
    scratch_types=[
        pltpu.VMEM((128, 128), _i32),
        pltpu.VMEM((8, 128), _i32),
        pltpu.VMEM((128, K), jnp.float32),
        pltpu.SemaphoreType.DMA,
    ],
)
def _unscramble_sc(urows_hbm, ujpos_hbm, ucnt_hbm, irows_hbm, ijpos_hbm,
                   icnt_hbm, ux_hbm, ix_hbm, jv_v, cb_v, rb_v, sem):
    wid = lax.axis_index("s") * NC + lax.axis_index("c")

    def phase(rows_hbm, jpos_hbm, cnt_hbm, out_hbm):
        pltpu.sync_copy(cnt_hbm.at[pl.ds(wid * 8, 8)], cb_v)
        cnt = cb_v[0, :][0]
        pltpu.sync_copy(jpos_hbm.at[wid], jv_v)
        nch = (cnt + 127) // 128

        def chunk(c, _):
            pltpu.sync_copy(rows_hbm.at[pl.ds(wid * B + c * 128, 128)], rb_v)
            pltpu.async_copy(rb_v, out_hbm.at[jv_v.at[c]], sem).wait()
            return _

        lax.fori_loop(0, nch, chunk, 0)

    phase(urows_hbm, ujpos_hbm, ucnt_hbm, ux_hbm)
    phase(irows_hbm, ijpos_hbm, icnt_hbm, ix_hbm)


def _mlp_body(ux_ref, ix_ref, w1u_ref, w1i_ref, b1_ref, w2_ref, b2_ref,
              woutt_ref, bout_ref, out_ref):
    bf = jnp.bfloat16
    h = jnp.dot(ux_ref[...].astype(bf), w1u_ref[...].astype(bf),
                preferred_element_type=jnp.float32)
    h = h + jnp.dot(ix_ref[...].astype(bf), w1i_ref[...].astype(bf),
                    preferred_element_type=jnp.float32)
    h = jnp.maximum(h + b1_ref[...], 0.0)
    h = jnp.maximum(
        jnp.dot(h.astype(bf), w2_ref[...].astype(bf),
                preferred_element_type=jnp.float32) + b2_ref[...], 0.0)
    o = jnp.sum(h.astype(bf).astype(jnp.float32) * woutt_ref[...], axis=1,
                keepdims=True) + bout_ref[...]
    out_ref[...] = jax.nn.sigmoid(o) * RATING_RANGE + LOWEST_RATING


def kernel(users, items, U, I, W1, b1, W2, b2, Wout, bout):
    users = users.astype(_i32)
    items = items.astype(_i32)
    ut = U.T
    it = I.T
    utail = U[TAIL0:, :].T
    itail = I[TAIL0:, :].T
    ur, uj, uc, ir, ij, ic = _scan_sc(users, items, ut, it, utail, itail)
    ux, ix = _unscramble_sc(ur.reshape(NW * B, K), uj, uc,
                            ir.reshape(NW * B, K), ij, ic)

    w1u = W1[:K, :]
    w1i = W1[K:, :]
    bb = 4096
    full = lambda shape: pl.BlockSpec(shape, lambda i: (0, 0))
    out = pl.pallas_call(
        _mlp_body,
        grid=(B // bb,),
        in_specs=[
            pl.BlockSpec((bb, K), lambda i: (i, 0)),
            pl.BlockSpec((bb, K), lambda i: (i, 0)),
            full((K, 64)),
            full((K, 64)),
            full((1, 64)),
            full((64, K)),
            full((1, K)),
            full((1, K)),
            full((1, 1)),
        ],
        out_specs=pl.BlockSpec((bb, 1), lambda i: (i, 0)),
        out_shape=jax.ShapeDtypeStruct((B, 1), jnp.float32),
    )(ux, ix, w1u, w1i, b1.reshape(1, -1), W2, b2.reshape(1, -1),
      Wout.reshape(1, -1), bout.reshape(1, 1))
    return out
